# baseline (speedup 1.0000x reference)
import functools

import jax
import jax.numpy as jnp
from jax import lax
from jax.experimental import pallas as pl
from jax.experimental.pallas import tpu as pltpu
from jax.experimental.pallas import tpu_sc as plsc

B = 4096
L = 50
D = 32
N = B * L

NC = 2
NS = 16
NW = NC * NS

N_PER_W = N // NW
CHUNK = 128
N_CHUNKS = N_PER_W // CHUNK


def _make_kernel():
    mesh = plsc.VectorSubcoreMesh(core_axis_name="c", subcore_axis_name="s")

    @functools.partial(
        pl.kernel,
        out_type=jax.ShapeDtypeStruct((N, D), jnp.float32),
        mesh=mesh,
        scratch_types=[
            pltpu.VMEM((N_PER_W,), jnp.int32),
            pltpu.VMEM((2, CHUNK, D), jnp.float32),
            pltpu.SemaphoreType.DMA,
            pltpu.SemaphoreType.DMA,
        ],
        compiler_params=pltpu.CompilerParams(use_tc_tiling_on_sc=False),
    )
    def gather_kernel(keys_hbm, table_hbm, out_hbm, idx_v, rows_v, sem0, sem1):
        wid = lax.axis_index("s") * NC + lax.axis_index("c")
        base = wid * N_PER_W
        sems = (sem0, sem1)

        pltpu.sync_copy(keys_hbm.at[pl.ds(base, N_PER_W)], idx_v)

        def start_gather(c, buf):
            pltpu.async_copy(
                table_hbm.at[idx_v.at[pl.ds(c * CHUNK, CHUNK)]],
                rows_v.at[buf],
                sems[buf],
            )

        def wait_and_flush(c, buf):
            pltpu.make_async_copy(
                table_hbm.at[idx_v.at[pl.ds(c * CHUNK, CHUNK)]],
                rows_v.at[buf],
                sems[buf],
            ).wait()
            pltpu.sync_copy(
                rows_v.at[buf], out_hbm.at[pl.ds(base + c * CHUNK, CHUNK)]
            )

        start_gather(0, 0)

        def body(c, _):
            start_gather(c + 1, 1)
            wait_and_flush(c, 0)
            start_gather(c + 2, 0)
            wait_and_flush(c + 1, 1)
            return _

        lax.fori_loop(0, (N_CHUNKS - 2) // 2, lambda i, c: body(2 * i, c), ())
        start_gather(N_CHUNKS - 1, 1)
        wait_and_flush(N_CHUNKS - 2, 0)
        wait_and_flush(N_CHUNKS - 1, 1)

    return gather_kernel


_gather = _make_kernel()


@jax.jit
def kernel(keys, table):
    flat_keys = keys.reshape((N,)).astype(jnp.int32)
    out = _gather(flat_keys, table)
    return out.reshape((B, L, D))


# speedup vs baseline: 1.0159x; 1.0159x over previous
import functools

import jax
import jax.numpy as jnp
from jax import lax
from jax.experimental import pallas as pl
from jax.experimental.pallas import tpu as pltpu
from jax.experimental.pallas import tpu_sc as plsc

B = 4096
L = 50
D = 32
N = B * L

NC = 2
NS = 16
NW = NC * NS

N_PER_W = N // NW
CHUNK = 640
NBUF = 2
N_CHUNKS = N_PER_W // CHUNK
assert N_PER_W % CHUNK == 0
assert N_CHUNKS % NBUF == 0 and N_CHUNKS >= 2 * NBUF
assert CHUNK % 8 == 0


def _make_kernel():
    mesh = plsc.VectorSubcoreMesh(core_axis_name="c", subcore_axis_name="s")

    @functools.partial(
        pl.kernel,
        out_type=jax.ShapeDtypeStruct((N, D), jnp.float32),
        mesh=mesh,
        scratch_types=[
            pltpu.VMEM((N_PER_W,), jnp.int32),
            pltpu.VMEM((NBUF, CHUNK, D), jnp.float32),
            [pltpu.SemaphoreType.DMA] * NBUF,
        ],
        compiler_params=pltpu.CompilerParams(use_tc_tiling_on_sc=False),
    )
    def gather_kernel(keys_hbm, table_hbm, out_hbm, idx_v, rows_v, sems):
        wid = lax.axis_index("s") * NC + lax.axis_index("c")
        base = wid * N_PER_W

        pltpu.sync_copy(keys_hbm.at[pl.ds(base, N_PER_W)], idx_v)

        def start_gather(c, buf):
            pltpu.async_copy(
                table_hbm.at[idx_v.at[pl.ds(c * CHUNK, CHUNK)]],
                rows_v.at[buf],
                sems[buf],
            )

        def wait_and_flush(c, buf):
            pltpu.make_async_copy(
                table_hbm.at[idx_v.at[pl.ds(c * CHUNK, CHUNK)]],
                rows_v.at[buf],
                sems[buf],
            ).wait()
            pltpu.sync_copy(
                rows_v.at[buf], out_hbm.at[pl.ds(base + c * CHUNK, CHUNK)]
            )

        for b in range(NBUF):
            start_gather(b, b)

        def body(i, _):
            c0 = i * NBUF
            for b in range(NBUF):
                wait_and_flush(c0 + b, b)
                start_gather(c0 + NBUF + b, b)
            return _

        lax.fori_loop(0, N_CHUNKS // NBUF - 1, body, ())
        for b in range(NBUF):
            wait_and_flush(N_CHUNKS - NBUF + b, b)

    return gather_kernel


_gather = _make_kernel()


@jax.jit
def kernel(keys, table):
    flat_keys = keys.reshape((N,)).astype(jnp.int32)
    out = _gather(flat_keys, table)
    return out.reshape((B, L, D))


# speedup vs baseline: 1.1877x; 1.1692x over previous
import functools

import jax
import jax.numpy as jnp
from jax import lax
from jax.experimental import pallas as pl
from jax.experimental.pallas import tpu as pltpu
from jax.experimental.pallas import tpu_sc as plsc

B = 4096
L = 50
D = 32
N = B * L
KR = N // 128
TR = 1000000
OR = N * D // 128

NC = 2
NS = 16
NW = NC * NS

RPW = KR // NW
CH = 128
NBUF = 2


def _make_kernel():
    mesh = plsc.VectorSubcoreMesh(core_axis_name="c", subcore_axis_name="s")

    @functools.partial(
        pl.kernel,
        out_type=jax.ShapeDtypeStruct((OR, 128), jnp.float32),
        mesh=mesh,
        scratch_types=[
            pltpu.VMEM((RPW, CH), jnp.int32),
            pltpu.VMEM((NBUF, CH, 128), jnp.float32),
            pltpu.VMEM((NBUF, CH // 4, 128), jnp.float32),
            pltpu.VMEM((64,), jnp.int32),
            [pltpu.SemaphoreType.DMA] * NBUF,
            [pltpu.SemaphoreType.DMA] * NBUF,
            pltpu.SemaphoreType.DMA,
        ],
    )
    def gather_kernel(keys_hbm, table_hbm, out_hbm,
                      kv, rows, dst, kidx, gsems, wsems, ksem):
        wid = lax.axis_index("s") * NC + lax.axis_index("c")
        kbase = wid * RPW
        obase = wid * (RPW * 32)

        lane = jax.lax.iota(jnp.int32, 16)
        for g in range(4):
            kidx[pl.ds(g * 16, 16)] = lane + (kbase + g * 16)
        pltpu.async_copy(keys_hbm.at[kidx.at[pl.ds(0, RPW)]], kv, ksem).wait()

        def start_gather(c, buf):
            pltpu.async_copy(table_hbm.at[kv.at[c]], rows.at[buf],
                             gsems[buf])

        def wait_gather(c, buf):
            pltpu.make_async_copy(table_hbm.at[kv.at[c]], rows.at[buf],
                                  gsems[buf]).wait()

        def extract(buf):
            def grp(g, _):
                for t in range(4):
                    kk = g * 4 + t
                    dst[buf, g, pl.ds(t * 32, 16)] = rows[buf, kk, pl.ds(0, 16)]
                    dst[buf, g, pl.ds(t * 32 + 16, 16)] = (
                        rows[buf, kk, pl.ds(16, 16)])
                return _
            lax.fori_loop(0, CH // 4, grp, ())

        def start_write(c, buf):
            pltpu.async_copy(dst.at[buf],
                             out_hbm.at[pl.ds(obase + c * 32, 32)],
                             wsems[buf])

        def wait_write(c, buf):
            pltpu.make_async_copy(dst.at[buf],
                                  out_hbm.at[pl.ds(obase + c * 32, 32)],
                                  wsems[buf]).wait()

        for b in range(NBUF):
            start_gather(b, b)

        def step(i, _):
            for b in range(NBUF):
                c = i * NBUF + b
                wait_gather(c, b)

                @pl.when(c >= NBUF)
                def _reclaim():
                    wait_write(c - NBUF, b)

                extract(b)
                start_write(c, b)

                @pl.when(c + NBUF < RPW)
                def _next():
                    start_gather(c + NBUF, b)
            return _

        lax.fori_loop(0, RPW // NBUF, step, ())
        wait_write(RPW - 2, 0)
        wait_write(RPW - 1, 1)

    return gather_kernel


_gather = _make_kernel()


@jax.jit
def kernel(keys, table):
    keys_r = keys.astype(jnp.int32).reshape((KR, 128))
    table_r = jnp.pad(table, ((0, 0), (0, 128 - D)))
    out = _gather(keys_r, table_r)
    return out.reshape((B, L, D))
